# baseline (speedup 1.0000x reference)
import functools

import jax
import jax.numpy as jnp
from jax import lax
from jax.experimental import pallas as pl
from jax.experimental.pallas import tpu as pltpu
from jax.experimental.pallas import tpu_sc as plsc

VOCAB = 10
HIDDEN = 128
PAIR = 2
D = PAIR * HIDDEN
NC, NS = 2, 16
NW = NC * NS
CHUNK = 128


@functools.partial(jax.jit, static_argnames=("nchunks",))
def _emb_lookup(idx, table, nchunks):
    @functools.partial(
        pl.kernel,
        out_type=jax.ShapeDtypeStruct((NW * nchunks * CHUNK, PAIR, HIDDEN), jnp.float32),
        mesh=plsc.VectorSubcoreMesh(core_axis_name="c", subcore_axis_name="s"),
        scratch_types=[
            pltpu.VMEM((nchunks, CHUNK), jnp.int32),
            pltpu.VMEM((2, CHUNK, PAIR, HIDDEN), jnp.float32),
            pltpu.VMEM_SHARED((VOCAB * VOCAB, PAIR, HIDDEN), jnp.float32),
            pltpu.SemaphoreType.DMA,
            pltpu.SemaphoreType.DMA,
            pltpu.SemaphoreType.DMA,
            pltpu.SemaphoreType.DMA,
        ],
    )
    def k(idx_hbm, table_hbm, out_hbm, idx_v, rbuf, table_sp, gs0, gs1, ws0, ws1):
        wid = lax.axis_index("s") * NC + lax.axis_index("c")

        @pl.when(lax.axis_index("s") == 0)
        def _():
            pltpu.sync_copy(table_hbm, table_sp)

        pltpu.sync_copy(idx_hbm.at[wid], idx_v)
        plsc.subcore_barrier()
        gs = (gs0, gs1)
        ws = (ws0, ws1)

        def start_gather(j, b):
            pltpu.async_copy(table_sp.at[idx_v.at[j]], rbuf.at[b], gs[b])

        def wait_gather(b):
            pltpu.make_async_copy(table_sp, rbuf.at[b], gs[b]).wait()

        def out_slice(j):
            return out_hbm.at[pl.ds((wid * nchunks + j) * CHUNK, CHUNK)]

        def wait_write(j, b):
            pltpu.make_async_copy(rbuf.at[b], out_slice(j), ws[b]).wait()

        start_gather(0, 0)
        start_gather(1, 1)

        def body(i, carry):
            for b in range(2):
                j = i * 2 + b
                wait_gather(b)
                pltpu.async_copy(rbuf.at[b], out_slice(j), ws[b])

                @pl.when(j + 2 < nchunks)
                def _():
                    wait_write(j, b)
                    start_gather(j + 2, b)

            return carry

        lax.fori_loop(0, nchunks // 2, body, 0)
        wait_write(nchunks - 2, 0)
        wait_write(nchunks - 1, 1)

    return k(idx, table)


def kernel(input_ids, word_embeddings):
    b, s = input_ids.shape
    n = b * s
    npairs = n // PAIR
    nchunks = npairs // (NW * CHUNK)
    assert npairs == NW * CHUNK * nchunks and nchunks % 2 == 0
    ids = input_ids.reshape(npairs, PAIR).astype(jnp.int32)
    pair_idx = (ids[:, 0] * VOCAB + ids[:, 1]).reshape(NW, nchunks, CHUNK)
    pair_table = jnp.stack(
        [
            jnp.repeat(word_embeddings, VOCAB, axis=0),
            jnp.tile(word_embeddings, (VOCAB, 1)),
        ],
        axis=1,
    )
    out = _emb_lookup(pair_idx, pair_table, nchunks)
    return out.reshape(b, s, HIDDEN)


# speedup vs baseline: 1.9992x; 1.9992x over previous
import functools

import jax
import jax.numpy as jnp
from jax import lax
from jax.experimental import pallas as pl
from jax.experimental.pallas import tpu as pltpu
from jax.experimental.pallas import tpu_sc as plsc

VOCAB = 10
HIDDEN = 128
NC, NS = 2, 16
NW = NC * NS
CHUNK = 128
K = 2
ROWS = K * CHUNK


@functools.partial(jax.jit, static_argnames=("nidx",))
def _emb_lookup(idx, table, nidx):
    nchunks = nidx // K

    @functools.partial(
        pl.kernel,
        out_type=jax.ShapeDtypeStruct((NW * nidx * CHUNK, HIDDEN), jnp.float32),
        mesh=plsc.VectorSubcoreMesh(core_axis_name="c", subcore_axis_name="s"),
        scratch_types=[
            pltpu.VMEM((nidx, CHUNK), jnp.int32),
            pltpu.VMEM((2, ROWS, HIDDEN), jnp.float32),
            pltpu.VMEM_SHARED((VOCAB, HIDDEN), jnp.float32),
            pltpu.SemaphoreType.DMA,
            pltpu.SemaphoreType.DMA,
            pltpu.SemaphoreType.DMA,
            pltpu.SemaphoreType.DMA,
        ],
    )
    def k(idx_hbm, table_hbm, out_hbm, idx_v, rbuf, table_sp, gs0, gs1, ws0, ws1):
        wid = lax.axis_index("s") * NC + lax.axis_index("c")

        @pl.when(lax.axis_index("s") == 0)
        def _():
            pltpu.sync_copy(table_hbm, table_sp)

        pltpu.sync_copy(idx_hbm.at[wid], idx_v)
        plsc.subcore_barrier()
        gs = (gs0, gs1)
        ws = (ws0, ws1)

        def start_gather(j, b):
            for t in range(K):
                pltpu.async_copy(
                    table_sp.at[idx_v.at[j * K + t]],
                    rbuf.at[b, pl.ds(t * CHUNK, CHUNK)],
                    gs[b],
                )

        def wait_gather(b):
            for t in range(K):
                pltpu.make_async_copy(
                    table_sp, rbuf.at[b, pl.ds(t * CHUNK, CHUNK)], gs[b]
                ).wait()

        def out_slice(j):
            return out_hbm.at[pl.ds((wid * nchunks + j) * ROWS, ROWS)]

        def wait_write(j, b):
            pltpu.make_async_copy(rbuf.at[b], out_slice(j), ws[b]).wait()

        start_gather(0, 0)
        start_gather(1, 1)

        def body(i, carry):
            for b in range(2):
                j = i * 2 + b
                wait_gather(b)
                pltpu.async_copy(rbuf.at[b], out_slice(j), ws[b])

                @pl.when(j + 2 < nchunks)
                def _():
                    wait_write(j, b)
                    start_gather(j + 2, b)

            return carry

        lax.fori_loop(0, nchunks // 2, body, 0)
        wait_write(nchunks - 2, 0)
        wait_write(nchunks - 1, 1)

    return k(idx, table)


def kernel(input_ids, word_embeddings):
    b, s = input_ids.shape
    n = b * s
    assert n % (NW * CHUNK * K) == 0
    nidx = n // (NW * CHUNK)
    idx = input_ids.reshape(NW, nidx, CHUNK).astype(jnp.int32)
    out = _emb_lookup(idx, word_embeddings, nidx)
    return out.reshape(b, s, HIDDEN)
